# baseline (speedup 1.0000x reference)
import jax
import jax.numpy as jnp
from jax.experimental import pallas as pl
from jax.experimental.pallas import tpu as pltpu

_N = 100000
_B = 1024
_D = 64
_BN = 1024
_T = 1.0


def _fused_body(x_ref, t_ref, v_ref, out_ref, loss_ref, m_ref, s_ref, ll_ref):
    j = pl.program_id(0)
    nj = pl.num_programs(0)

    @pl.when(j == 0)
    def _init():
        m_ref[...] = jnp.full_like(m_ref, -jnp.inf)
        s_ref[...] = jnp.zeros_like(s_ref)
        ll_ref[...] = jnp.zeros_like(ll_ref)

    x = x_ref[...]
    v = v_ref[...]
    block = jax.lax.dot_general(
        x, v, (((1,), (1,)), ((), ())),
        preferred_element_type=jnp.float32)
    if _T != 1.0:
        block = block * _T
    out_ref[...] = block

    cols = j * _BN + jax.lax.broadcasted_iota(jnp.int32, (_B, _BN), 1)
    valid = cols < _N
    t = t_ref[...]

    bm = jnp.max(jnp.where(valid, block, -jnp.inf), axis=1, keepdims=True)
    m_old = m_ref[...]
    m_new = jnp.maximum(m_old, bm)
    e = jnp.where(valid, jnp.exp(block - m_new), 0.0)
    s_ref[...] = s_ref[...] * jnp.exp(m_old - m_new) + jnp.sum(
        e, axis=1, keepdims=True)
    m_ref[...] = m_new
    ll_ref[...] = ll_ref[...] + jnp.sum(
        jnp.where(cols == t, block, 0.0), axis=1, keepdims=True)

    @pl.when(j == nj - 1)
    def _fin():
        logz = m_ref[...] + jnp.log(s_ref[...])
        loss_ref[0, 0] = jnp.sum(logz - ll_ref[...]) / _B


def _fused_call(inputs, targets2d, V, interpret=False):
    grid = (pl.cdiv(_N, _BN),)
    return pl.pallas_call(
        _fused_body,
        grid=grid,
        in_specs=[
            pl.BlockSpec((_B, _D), lambda j: (0, 0)),
            pl.BlockSpec((_B, 1), lambda j: (0, 0)),
            pl.BlockSpec((_BN, _D), lambda j: (j, 0)),
        ],
        out_specs=[
            pl.BlockSpec((_B, _BN), lambda j: (0, j)),
            pl.BlockSpec(memory_space=pltpu.SMEM),
        ],
        out_shape=[
            jax.ShapeDtypeStruct((_B, _N), jnp.float32),
            jax.ShapeDtypeStruct((1, 1), jnp.float32),
        ],
        scratch_shapes=[
            pltpu.VMEM((_B, 1), jnp.float32),
            pltpu.VMEM((_B, 1), jnp.float32),
            pltpu.VMEM((_B, 1), jnp.float32),
        ],
        compiler_params=pltpu.CompilerParams(
            dimension_semantics=("arbitrary",)),
        interpret=interpret,
    )(inputs, targets2d, V)


def kernel(inputs, targets, V):
    t2d = targets.astype(jnp.int32).reshape(_B, 1)
    outputs, loss = _fused_call(inputs, t2d, V)
    return (loss[0, 0], outputs)


# speedup vs baseline: 1.2154x; 1.2154x over previous
import jax
import jax.numpy as jnp
from jax.experimental import pallas as pl
from jax.experimental.pallas import tpu as pltpu

_N = 100000
_B = 1024
_D = 64
_BN = 2048
_T = 1.0


def _fused_body(x_ref, t_ref, v_ref, out_ref, loss_ref, m_ref, s_ref, ll_ref):
    j = pl.program_id(0)
    nj = pl.num_programs(0)

    x = x_ref[...]
    v = v_ref[...]

    @pl.when(j == 0)
    def _init():
        xf = x.astype(jnp.float32)
        m_ref[...] = jnp.sqrt(jnp.sum(xf * xf, axis=1, keepdims=True)) + 1e-3
        s_ref[...] = jnp.zeros_like(s_ref)
        ll_ref[...] = jnp.zeros_like(ll_ref)

    block = jax.lax.dot_general(
        x, v, (((1,), (1,)), ((), ())),
        preferred_element_type=jnp.float32)
    if _T != 1.0:
        block = block * _T
    out_ref[...] = block

    t = t_ref[...]
    m = m_ref[...]
    cols = j * _BN + jax.lax.broadcasted_iota(jnp.int32, (_B, _BN), 1)
    eq = cols == t

    @pl.when(j < nj - 1)
    def _full_block():
        e = jnp.exp(block - m)
        s_ref[...] = s_ref[...] + jnp.sum(e, axis=1, keepdims=True)
        ll_ref[...] = ll_ref[...] + jnp.sum(
            jnp.where(eq, block, 0.0), axis=1, keepdims=True)

    @pl.when(j == nj - 1)
    def _ragged_block():
        valid = cols < _N
        e = jnp.where(valid, jnp.exp(block - m), 0.0)
        s = s_ref[...] + jnp.sum(e, axis=1, keepdims=True)
        ll = ll_ref[...] + jnp.sum(
            jnp.where(eq, block, 0.0), axis=1, keepdims=True)
        logz = m + jnp.log(s)
        loss_ref[0, 0] = jnp.sum(logz - ll) / _B


def _fused_call(inputs_bf, targets2d, v_bf, interpret=False):
    grid = (pl.cdiv(_N, _BN),)
    return pl.pallas_call(
        _fused_body,
        grid=grid,
        in_specs=[
            pl.BlockSpec((_B, _D), lambda j: (0, 0)),
            pl.BlockSpec((_B, 1), lambda j: (0, 0)),
            pl.BlockSpec((_BN, _D), lambda j: (j, 0)),
        ],
        out_specs=[
            pl.BlockSpec((_B, _BN), lambda j: (0, j)),
            pl.BlockSpec(memory_space=pltpu.SMEM),
        ],
        out_shape=[
            jax.ShapeDtypeStruct((_B, _N), jnp.float32),
            jax.ShapeDtypeStruct((1, 1), jnp.float32),
        ],
        scratch_shapes=[
            pltpu.VMEM((_B, 1), jnp.float32),
            pltpu.VMEM((_B, 1), jnp.float32),
            pltpu.VMEM((_B, 1), jnp.float32),
        ],
        compiler_params=pltpu.CompilerParams(
            dimension_semantics=("arbitrary",)),
        interpret=interpret,
    )(inputs_bf, targets2d, v_bf)


def kernel(inputs, targets, V):
    t2d = targets.astype(jnp.int32).reshape(_B, 1)
    outputs, loss = _fused_call(
        inputs.astype(jnp.bfloat16), t2d, V.astype(jnp.bfloat16))
    return (loss[0, 0], outputs)
